# Initial kernel scaffold; baseline (speedup 1.0000x reference)
import jax
import jax.numpy as jnp
from jax.experimental import pallas as pl


def kernel(x, W1, W2, W3, W4, W5, W6, W7):
    raise NotImplementedError("write your pallas kernel here")


# speedup vs baseline: 1.0077x; 1.0077x over previous
import functools

import jax
import jax.numpy as jnp
import numpy as np
from jax.experimental import pallas as pl

K = 20


def _conv_stats_body(h_ref, w_ref, y_ref, s_ref):
    h = h_ref[0]
    w = w_ref[...]
    y = jnp.dot(w, h, preferred_element_type=jnp.float32)
    y_ref[0] = y
    s_ref[0, 0, :] = jnp.sum(y, axis=1)
    s_ref[0, 1, :] = jnp.sum(y * y, axis=1)


def _norm_conv_stats_body(y_ref, m_ref, v_ref, w_ref, z_ref, s_ref):
    y = y_ref[0]
    m = m_ref[...][:, None]
    v = v_ref[...][:, None]
    zn = (y - m) * jax.lax.rsqrt(v + 1e-5)
    zn = jnp.where(zn >= 0, zn, 0.2 * zn)
    z = jnp.dot(w_ref[...], zn, preferred_element_type=jnp.float32)
    z_ref[0] = z
    s_ref[0, 0, :] = jnp.sum(z, axis=1)
    s_ref[0, 1, :] = jnp.sum(z * z, axis=1)


def _norm_conv_body(y_ref, m_ref, v_ref, w_ref, z_ref):
    y = y_ref[0]
    m = m_ref[...][:, None]
    v = v_ref[...][:, None]
    zn = (y - m) * jax.lax.rsqrt(v + 1e-5)
    zn = jnp.where(zn >= 0, zn, 0.2 * zn)
    z_ref[0] = jnp.dot(w_ref[...], zn, preferred_element_type=jnp.float32)


def _conv_stats(h, w, nb=1024):
    B, C, N = h.shape
    O = w.shape[0]
    grid = (B, N // nb)
    y, s = pl.pallas_call(
        _conv_stats_body,
        grid=grid,
        in_specs=[
            pl.BlockSpec((1, C, nb), lambda b, n: (b, 0, n)),
            pl.BlockSpec((O, C), lambda b, n: (0, 0)),
        ],
        out_specs=[
            pl.BlockSpec((1, O, nb), lambda b, n: (b, 0, n)),
            pl.BlockSpec((1, 2, O), lambda b, n: (b * (N // nb) + n, 0, 0)),
        ],
        out_shape=[
            jax.ShapeDtypeStruct((B, O, N), jnp.float32),
            jax.ShapeDtypeStruct((B * (N // nb), 2, O), jnp.float32),
        ],
    )(h, w)
    tot = jnp.sum(s, axis=0)
    cnt = B * N
    mean = tot[0] / cnt
    var = tot[1] / cnt - mean * mean
    return y, mean, var


def _norm_conv_stats(y, mean, var, w, nb=1024):
    B, C, N = y.shape
    O = w.shape[0]
    grid = (B, N // nb)
    z, s = pl.pallas_call(
        _norm_conv_stats_body,
        grid=grid,
        in_specs=[
            pl.BlockSpec((1, C, nb), lambda b, n: (b, 0, n)),
            pl.BlockSpec((C,), lambda b, n: (0,)),
            pl.BlockSpec((C,), lambda b, n: (0,)),
            pl.BlockSpec((O, C), lambda b, n: (0, 0)),
        ],
        out_specs=[
            pl.BlockSpec((1, O, nb), lambda b, n: (b, 0, n)),
            pl.BlockSpec((1, 2, O), lambda b, n: (b * (N // nb) + n, 0, 0)),
        ],
        out_shape=[
            jax.ShapeDtypeStruct((B, O, N), jnp.float32),
            jax.ShapeDtypeStruct((B * (N // nb), 2, O), jnp.float32),
        ],
    )(y, mean, var, w)
    tot = jnp.sum(s, axis=0)
    cnt = B * N
    m = tot[0] / cnt
    v = tot[1] / cnt - m * m
    return z, m, v


def _norm_conv(y, mean, var, w, nb=1024):
    B, C, N = y.shape
    O = w.shape[0]
    grid = (B, N // nb)
    return pl.pallas_call(
        _norm_conv_body,
        grid=grid,
        in_specs=[
            pl.BlockSpec((1, C, nb), lambda b, n: (b, 0, n)),
            pl.BlockSpec((C,), lambda b, n: (0,)),
            pl.BlockSpec((C,), lambda b, n: (0,)),
            pl.BlockSpec((O, C), lambda b, n: (0, 0)),
        ],
        out_specs=pl.BlockSpec((1, O, nb), lambda b, n: (b, 0, n)),
        out_shape=jax.ShapeDtypeStruct((B, O, N), jnp.float32),
    )(y, mean, var, w)


def _knn(x, k):
    xx = jnp.sum(x * x, axis=1)
    inner = jnp.einsum('bcn,bcm->bnm', x, x)
    neg_d2 = -(xx[:, :, None] - 2.0 * inner + xx[:, None, :])
    return jax.lax.top_k(neg_d2, k)[1]


def _gather_nb(x, idx):
    return jax.vmap(lambda xb, ib: xb[:, ib])(x, idx)


def _gather_op(x, idx):
    return jax.vmap(lambda xb, ib: xb[:, ib])(x, idx)


def _group_layer(x, idx):
    nb = _gather_nb(x, idx)
    xc = jnp.broadcast_to(x[:, :, :, None], nb.shape)
    return jnp.concatenate([nb - xc, xc], axis=1)


def _eigen_graph(x, k):
    idx_EU = _knn(x, k)
    nb = _gather_nb(x, idx_EU)
    nb = jnp.transpose(nb, (0, 2, 3, 1))
    nb = nb - jnp.mean(nb, axis=2, keepdims=True)
    cov = jnp.einsum('bnki,bnkj->bnij', nb, nb) / k
    eig = jnp.linalg.eigvalsh(jax.lax.stop_gradient(cov))
    idx_EI = _knn(jnp.transpose(eig, (0, 2, 1)), k)
    return x, idx_EU, idx_EI


def _graph_distance(x, idx):
    nb = _gather_nb(x, idx)
    return jnp.sqrt(jnp.sum((nb - x[:, :, :, None]) ** 2, axis=1, keepdims=True) + 1e-12)


def _bn(x, axes):
    m = jnp.mean(x, axis=axes, keepdims=True)
    v = jnp.var(x, axis=axes, keepdims=True)
    return (x - m) / jnp.sqrt(v + 1e-5)


def _lrelu(x):
    return jax.nn.leaky_relu(x, 0.2)


def _conv2d(W, x):
    return jnp.einsum('oc,bcnk->bonk', W, x)


def _fps(xyz, npoint):
    xyz = jax.lax.stop_gradient(xyz)
    B, N, _ = xyz.shape
    def body(i, state):
        idxs, dists, far = state
        idxs = idxs.at[:, i].set(far)
        centroid = jnp.take_along_axis(xyz, far[:, None, None], axis=1)
        d = jnp.sum((xyz - centroid) ** 2, axis=-1)
        dists = jnp.minimum(dists, d)
        far = jnp.argmax(dists, axis=-1).astype(jnp.int32)
        return idxs, dists, far
    idxs0 = jnp.zeros((B, npoint), dtype=jnp.int32)
    d0 = jnp.full((B, N), 1e10, dtype=xyz.dtype)
    f0 = jnp.zeros((B,), dtype=jnp.int32)
    idxs, _, _ = jax.lax.fori_loop(0, npoint, body, (idxs0, d0, f0))
    return idxs


def _three_nn(unknown, known):
    aa = jnp.sum(unknown * unknown, axis=-1)[:, :, None]
    bb = jnp.sum(known * known, axis=-1)[:, None, :]
    ab = jnp.einsum('bnd,bmd->bnm', unknown, known)
    d2 = jnp.maximum(aa + bb - 2.0 * ab, 0.0)
    neg, idx = jax.lax.top_k(-d2, 3)
    dist = jnp.sqrt(jnp.maximum(-neg, 0.0) + 1e-12)
    return jax.lax.stop_gradient(dist), idx


def _three_interpolate(feats, idx, weight):
    def one(fb, ib, wb):
        nb = fb[:, ib]
        return jnp.sum(nb * wb[None], axis=-1)
    return jax.vmap(one)(feats, idx, weight)


def _gscm_first(points_t, k, W):
    xg, idx_EU, idx_EI = _eigen_graph(points_t, k)
    feat = jnp.concatenate([_group_layer(xg, idx_EU), _group_layer(xg, idx_EI)], axis=1)
    dist = _graph_distance(points_t, idx_EU)
    feat = jnp.concatenate([feat, dist], axis=1)
    y = _lrelu(_bn(_conv2d(W, feat), (0, 2, 3)))
    return jnp.max(y, axis=-1)


def _gscm(points_t, feats, k, W):
    _, idx_EU, idx_EI = _eigen_graph(points_t, k)
    feat = jnp.concatenate([_group_layer(feats, idx_EU), _group_layer(feats, idx_EI)], axis=1)
    y = _lrelu(_bn(_conv2d(W, feat), (0, 2, 3)))
    return jnp.max(y, axis=-1)


def kernel(x, W1, W2, W3, W4, W5, W6, W7):
    B, C, N = x.shape
    N2 = N // 2
    N3 = N // 4
    pts1 = jnp.transpose(x, (0, 2, 1))[:, :, :3]
    pts1_t = jnp.transpose(pts1, (0, 2, 1))
    x1 = _gscm_first(pts1_t, K, W1)
    fps2 = _fps(pts1, N2)
    pts2 = jnp.transpose(_gather_op(pts1_t, fps2), (0, 2, 1))
    x1_ds = _gather_op(x1, fps2)
    pts2_t = jnp.transpose(pts2, (0, 2, 1))
    x2 = _gscm(pts2_t, x1_ds, K, W2)
    fps3 = _fps(pts2, N3)
    pts3 = jnp.transpose(_gather_op(pts2_t, fps3), (0, 2, 1))
    x2_ds = _gather_op(x2, fps3)
    x1_ds = _gather_op(x1_ds, fps3)
    pts3_t = jnp.transpose(pts3, (0, 2, 1))
    x3 = _gscm(pts3_t, x2_ds, K, W3)
    h = jnp.concatenate([x1_ds, x2_ds, x3], axis=1)

    y4, m4, v4 = _conv_stats(h, W4, nb=1024)
    h = _lrelu((y4 - m4[None, :, None]) * jax.lax.rsqrt(v4[None, :, None] + 1e-5))

    dist, idx = _three_nn(pts2, pts3)
    w = 1.0 / (dist + 1e-8)
    w = w / jnp.sum(w, axis=2, keepdims=True)
    h = _three_interpolate(h, idx, w)
    dist, idx = _three_nn(pts1, pts2)
    w = 1.0 / (dist + 1e-8)
    w = w / jnp.sum(w, axis=2, keepdims=True)
    h = _three_interpolate(h, idx, w)

    y5, m5, v5 = _conv_stats(h, W5, nb=1024)
    y6, m6, v6 = _norm_conv_stats(y5, m5, v5, W6, nb=1024)
    out = _norm_conv(y6, m6, v6, W7, nb=1024)
    return out


# speedup vs baseline: 1.5238x; 1.5122x over previous
import functools

import jax
import jax.numpy as jnp
import numpy as np
from jax.experimental import pallas as pl

K = 20


def _conv_stats_body(h_ref, w_ref, y_ref, s_ref):
    h = h_ref[0]
    w = w_ref[...]
    y = jnp.dot(w, h, preferred_element_type=jnp.float32)
    y_ref[0] = y
    s_ref[0, 0, :] = jnp.sum(y, axis=1)
    s_ref[0, 1, :] = jnp.sum(y * y, axis=1)


def _norm_conv_stats_body(y_ref, m_ref, v_ref, w_ref, z_ref, s_ref):
    y = y_ref[0]
    m = m_ref[...][:, None]
    v = v_ref[...][:, None]
    zn = (y - m) * jax.lax.rsqrt(v + 1e-5)
    zn = jnp.where(zn >= 0, zn, 0.2 * zn)
    z = jnp.dot(w_ref[...], zn, preferred_element_type=jnp.float32)
    z_ref[0] = z
    s_ref[0, 0, :] = jnp.sum(z, axis=1)
    s_ref[0, 1, :] = jnp.sum(z * z, axis=1)


def _norm_conv_body(y_ref, m_ref, v_ref, w_ref, z_ref):
    y = y_ref[0]
    m = m_ref[...][:, None]
    v = v_ref[...][:, None]
    zn = (y - m) * jax.lax.rsqrt(v + 1e-5)
    zn = jnp.where(zn >= 0, zn, 0.2 * zn)
    z_ref[0] = jnp.dot(w_ref[...], zn, preferred_element_type=jnp.float32)


def _conv_stats(h, w, nb=1024):
    B, C, N = h.shape
    O = w.shape[0]
    grid = (B, N // nb)
    y, s = pl.pallas_call(
        _conv_stats_body,
        grid=grid,
        in_specs=[
            pl.BlockSpec((1, C, nb), lambda b, n: (b, 0, n)),
            pl.BlockSpec((O, C), lambda b, n: (0, 0)),
        ],
        out_specs=[
            pl.BlockSpec((1, O, nb), lambda b, n: (b, 0, n)),
            pl.BlockSpec((1, 2, O), lambda b, n: (b * (N // nb) + n, 0, 0)),
        ],
        out_shape=[
            jax.ShapeDtypeStruct((B, O, N), jnp.float32),
            jax.ShapeDtypeStruct((B * (N // nb), 2, O), jnp.float32),
        ],
    )(h, w)
    tot = jnp.sum(s, axis=0)
    cnt = B * N
    mean = tot[0] / cnt
    var = tot[1] / cnt - mean * mean
    return y, mean, var


def _norm_conv_stats(y, mean, var, w, nb=1024):
    B, C, N = y.shape
    O = w.shape[0]
    grid = (B, N // nb)
    z, s = pl.pallas_call(
        _norm_conv_stats_body,
        grid=grid,
        in_specs=[
            pl.BlockSpec((1, C, nb), lambda b, n: (b, 0, n)),
            pl.BlockSpec((C,), lambda b, n: (0,)),
            pl.BlockSpec((C,), lambda b, n: (0,)),
            pl.BlockSpec((O, C), lambda b, n: (0, 0)),
        ],
        out_specs=[
            pl.BlockSpec((1, O, nb), lambda b, n: (b, 0, n)),
            pl.BlockSpec((1, 2, O), lambda b, n: (b * (N // nb) + n, 0, 0)),
        ],
        out_shape=[
            jax.ShapeDtypeStruct((B, O, N), jnp.float32),
            jax.ShapeDtypeStruct((B * (N // nb), 2, O), jnp.float32),
        ],
    )(y, mean, var, w)
    tot = jnp.sum(s, axis=0)
    cnt = B * N
    m = tot[0] / cnt
    v = tot[1] / cnt - m * m
    return z, m, v


def _norm_conv(y, mean, var, w, nb=1024):
    B, C, N = y.shape
    O = w.shape[0]
    grid = (B, N // nb)
    return pl.pallas_call(
        _norm_conv_body,
        grid=grid,
        in_specs=[
            pl.BlockSpec((1, C, nb), lambda b, n: (b, 0, n)),
            pl.BlockSpec((C,), lambda b, n: (0,)),
            pl.BlockSpec((C,), lambda b, n: (0,)),
            pl.BlockSpec((O, C), lambda b, n: (0, 0)),
        ],
        out_specs=pl.BlockSpec((1, O, nb), lambda b, n: (b, 0, n)),
        out_shape=jax.ShapeDtypeStruct((B, O, N), jnp.float32),
    )(y, mean, var, w)


def _eig3_body(cov_ref, eig_ref):
    a00 = cov_ref[0, 0, :]
    a11 = cov_ref[0, 1, :]
    a22 = cov_ref[0, 2, :]
    a01 = cov_ref[0, 3, :]
    a02 = cov_ref[0, 4, :]
    a12 = cov_ref[0, 5, :]
    q = (a00 + a11 + a22) * (1.0 / 3.0)
    p1 = a01 * a01 + a02 * a02 + a12 * a12
    d01 = a00 - a11
    d02 = a00 - a22
    d12 = a11 - a22
    b00 = (d01 + d02) * (1.0 / 3.0)
    b11 = (-d01 + d12) * (1.0 / 3.0)
    b22 = (-d02 - d12) * (1.0 / 3.0)
    p2 = b00 * b00 + b11 * b11 + b22 * b22 + 2.0 * p1
    p = jnp.sqrt(p2 * (1.0 / 6.0) + 1e-30)
    detb = (b00 * (b11 * b22 - a12 * a12)
            - a01 * (a01 * b22 - a12 * a02)
            + a02 * (a01 * a12 - b11 * a02))
    r = detb / (2.0 * p * p * p)
    r = jnp.clip(r, -1.0, 1.0)

    def newton_root(rr):
        xx = jnp.ones_like(rr)
        for _ in range(25):
            g = (4.0 * xx * xx - 3.0) * xx - rr
            dg = jnp.maximum(12.0 * xx * xx - 3.0, 1e-8)
            xx = xx - g / dg
        return xx

    lmax = q + 2.0 * p * newton_root(r)
    lmin = q - 2.0 * p * newton_root(-r)
    lmid = 3.0 * q - lmax - lmin
    eig_ref[0, 0, :] = lmin
    eig_ref[0, 1, :] = lmid
    eig_ref[0, 2, :] = lmax


def _eig3(cov6):
    B, _, N = cov6.shape
    nb = min(N, 2048)
    return pl.pallas_call(
        _eig3_body,
        grid=(B, N // nb),
        in_specs=[pl.BlockSpec((1, 6, nb), lambda b, n: (b, 0, n))],
        out_specs=pl.BlockSpec((1, 3, nb), lambda b, n: (b, 0, n)),
        out_shape=jax.ShapeDtypeStruct((B, 3, N), jnp.float32),
    )(cov6)


def _fps_body(x_ref, y_ref, z_ref, idx_ref, npoint, rows, cols):
    X = x_ref[0]
    Y = y_ref[0]
    Z = z_ref[0]
    flat_i = (jax.lax.broadcasted_iota(jnp.int32, (rows, cols), 0) * cols
              + jax.lax.broadcasted_iota(jnp.int32, (rows, cols), 1))
    ocols = npoint // 8
    flat_o = (jax.lax.broadcasted_iota(jnp.int32, (8, ocols), 0) * ocols
              + jax.lax.broadcasted_iota(jnp.int32, (8, ocols), 1))

    def body(i, state):
        dists, far, acc = state
        acc = jnp.where(flat_o == i, far, acc)
        sel = flat_i == far
        cx = jnp.sum(jnp.where(sel, X, 0.0))
        cy = jnp.sum(jnp.where(sel, Y, 0.0))
        cz = jnp.sum(jnp.where(sel, Z, 0.0))
        dx = X - cx
        dy = Y - cy
        dz = Z - cz
        d = dx * dx + dy * dy + dz * dz
        dists = jnp.minimum(dists, d)
        m = jnp.max(dists)
        far2 = jnp.min(jnp.where(dists == m, flat_i, jnp.int32(2 ** 30)))
        return dists, far2, acc

    d0 = jnp.full((rows, cols), 1e10, dtype=jnp.float32)
    acc0 = jnp.zeros((8, ocols), dtype=jnp.int32)
    _, _, acc = jax.lax.fori_loop(0, npoint, body, (d0, jnp.int32(0), acc0))
    idx_ref[0] = acc


def _fps_pallas(pts, npoint):
    B, N, _ = pts.shape
    rows, cols = 8, N // 8
    xyz = jnp.transpose(pts, (0, 2, 1)).reshape(B, 3, rows, cols)
    body = functools.partial(_fps_body, npoint=npoint, rows=rows, cols=cols)
    return pl.pallas_call(
        body,
        grid=(B,),
        in_specs=[
            pl.BlockSpec((1, rows, cols), lambda b: (b, 0, 0))
            for _ in range(3)
        ],
        out_specs=pl.BlockSpec((1, 8, npoint // 8), lambda b: (b, 0, 0)),
        out_shape=jax.ShapeDtypeStruct((B, 8, npoint // 8), jnp.int32),
    )(xyz[:, 0], xyz[:, 1], xyz[:, 2]).reshape(B, npoint)


def _topk_body(rows_ref, cols_ref, idx_ref, val_ref, k, clamp):
    pb = rows_ref[0]
    pt = cols_ref[0]
    R = pb.shape[0]
    N = pt.shape[1]
    xxr = pb[:, 0] * pb[:, 0] + pb[:, 1] * pb[:, 1] + pb[:, 2] * pb[:, 2]
    xxm = pt[0, :] * pt[0, :] + pt[1, :] * pt[1, :] + pt[2, :] * pt[2, :]
    inner = jnp.dot(pb, pt, preferred_element_type=jnp.float32)
    if clamp:
        d2 = jnp.maximum((xxr[:, None] + xxm[None, :]) - 2.0 * inner, 0.0)
    else:
        d2 = (xxr[:, None] - 2.0 * inner) + xxm[None, :]
    ci = jax.lax.broadcasted_iota(jnp.int32, (R, N), 1)
    cols = []
    vals = []
    for _ in range(k):
        m = jnp.min(d2, axis=1)
        eq = d2 == m[:, None]
        col = jnp.min(jnp.where(eq, ci, jnp.int32(N)), axis=1)
        cols.append(col)
        vals.append(m)
        d2 = jnp.where(ci == col[:, None], jnp.float32(jnp.inf), d2)
    idx_ref[0] = jnp.stack(cols, axis=1)
    val_ref[0] = jnp.stack(vals, axis=1)


def _topk_pallas(rows, cols, k, clamp, rblk=512):
    B, Nr, _ = rows.shape
    Nc = cols.shape[2]
    rb = min(rblk, Nr)
    body = functools.partial(_topk_body, k=k, clamp=clamp)
    return pl.pallas_call(
        body,
        grid=(B, Nr // rb),
        in_specs=[
            pl.BlockSpec((1, rb, 3), lambda b, n: (b, n, 0)),
            pl.BlockSpec((1, 3, Nc), lambda b, n: (b, 0, 0)),
        ],
        out_specs=[
            pl.BlockSpec((1, rb, k), lambda b, n: (b, n, 0)),
            pl.BlockSpec((1, rb, k), lambda b, n: (b, n, 0)),
        ],
        out_shape=[
            jax.ShapeDtypeStruct((B, Nr, k), jnp.int32),
            jax.ShapeDtypeStruct((B, Nr, k), jnp.float32),
        ],
    )(rows, cols)


def _sel_body(s_ref, idx_ref, k):
    s = s_ref[0]
    R, N = s.shape
    ci = jax.lax.broadcasted_iota(jnp.int32, (R, N), 1)
    cols = []
    for _ in range(k):
        m = jnp.max(s, axis=1)
        col = jnp.min(jnp.where(s == m[:, None], ci, jnp.int32(N)), axis=1)
        cols.append(col)
        s = jnp.where(ci == col[:, None], -jnp.float32(jnp.inf), s)
    idx_ref[0] = jnp.stack(cols, axis=1)


def _topk_select(scores, k, rblk=512):
    B, Nr, Nc = scores.shape
    rb = min(rblk, Nr)
    return pl.pallas_call(
        functools.partial(_sel_body, k=k),
        grid=(B, Nr // rb),
        in_specs=[pl.BlockSpec((1, rb, Nc), lambda b, n: (b, n, 0))],
        out_specs=pl.BlockSpec((1, rb, k), lambda b, n: (b, n, 0)),
        out_shape=jax.ShapeDtypeStruct((B, Nr, k), jnp.int32),
    )(scores)


def _knn_exact(x, k):
    xx = jnp.sum(x * x, axis=1)
    inner = jnp.einsum('bcn,bcm->bnm', x, x)
    neg_d2 = -(xx[:, :, None] - 2.0 * inner + xx[:, None, :])
    return _topk_select(neg_d2, k)


def _knn(x, k):
    rows = jnp.transpose(x, (0, 2, 1))
    return _topk_pallas(rows, x, k, clamp=False)[0]


def _knn_with_d2(x, k):
    rows = jnp.transpose(x, (0, 2, 1))
    return _topk_pallas(rows, x, k, clamp=False)


def _gather_nb(x, idx):
    return jax.vmap(lambda xb, ib: xb[:, ib])(x, idx)


def _gather_op(x, idx):
    return jax.vmap(lambda xb, ib: xb[:, ib])(x, idx)


def _group_layer(x, idx):
    nb = _gather_nb(x, idx)
    xc = jnp.broadcast_to(x[:, :, :, None], nb.shape)
    return jnp.concatenate([nb - xc, xc], axis=1)


def _eigen_graph(x, k):
    idx_EU = _knn(x, k)
    nb = _gather_nb(x, idx_EU)
    nb = jnp.transpose(nb, (0, 2, 3, 1))
    nb = nb - jnp.mean(nb, axis=2, keepdims=True)
    cov = jnp.einsum('bnki,bnkj->bnij', nb, nb) / k
    eig = jnp.linalg.eigvalsh(jax.lax.stop_gradient(cov))
    idx_EI = _knn_exact(jnp.transpose(eig, (0, 2, 1)), k)
    return x, idx_EU, idx_EI


def _graph_distance(x, idx):
    nb = _gather_nb(x, idx)
    return jnp.sqrt(jnp.sum((nb - x[:, :, :, None]) ** 2, axis=1, keepdims=True) + 1e-12)


def _bn(x, axes):
    m = jnp.mean(x, axis=axes, keepdims=True)
    v = jnp.var(x, axis=axes, keepdims=True)
    return (x - m) / jnp.sqrt(v + 1e-5)


def _lrelu(x):
    return jax.nn.leaky_relu(x, 0.2)


def _conv2d(W, x):
    return jnp.einsum('oc,bcnk->bonk', W, x)


def _fps(xyz, npoint):
    xyz = jax.lax.stop_gradient(xyz)
    B, N, _ = xyz.shape
    def body(i, state):
        idxs, dists, far = state
        idxs = idxs.at[:, i].set(far)
        centroid = jnp.take_along_axis(xyz, far[:, None, None], axis=1)
        d = jnp.sum((xyz - centroid) ** 2, axis=-1)
        dists = jnp.minimum(dists, d)
        far = jnp.argmax(dists, axis=-1).astype(jnp.int32)
        return idxs, dists, far
    idxs0 = jnp.zeros((B, npoint), dtype=jnp.int32)
    d0 = jnp.full((B, N), 1e10, dtype=xyz.dtype)
    f0 = jnp.zeros((B,), dtype=jnp.int32)
    idxs, _, _ = jax.lax.fori_loop(0, npoint, body, (idxs0, d0, f0))
    return idxs


def _three_nn(unknown, known):
    idx, d2 = _topk_pallas(unknown, jnp.transpose(known, (0, 2, 1)), 3,
                           clamp=True)
    dist = jnp.sqrt(jnp.maximum(d2, 0.0) + 1e-12)
    return dist, idx


def _three_interpolate(feats, idx, weight):
    def one(fb, ib, wb):
        nb = fb[:, ib]
        return jnp.sum(nb * wb[None], axis=-1)
    return jax.vmap(one)(feats, idx, weight)


def _gscm_first(points_t, k, W):
    xg, idx_EU, idx_EI = _eigen_graph(points_t, k)
    feat = jnp.concatenate([_group_layer(xg, idx_EU), _group_layer(xg, idx_EI)], axis=1)
    dist = _graph_distance(points_t, idx_EU)
    feat = jnp.concatenate([feat, dist], axis=1)
    y = _lrelu(_bn(_conv2d(W, feat), (0, 2, 3)))
    return jnp.max(y, axis=-1)


def _gscm(points_t, feats, k, W):
    _, idx_EU, idx_EI = _eigen_graph(points_t, k)
    feat = jnp.concatenate([_group_layer(feats, idx_EU), _group_layer(feats, idx_EI)], axis=1)
    y = _lrelu(_bn(_conv2d(W, feat), (0, 2, 3)))
    return jnp.max(y, axis=-1)


def kernel(x, W1, W2, W3, W4, W5, W6, W7):
    B, C, N = x.shape
    N2 = N // 2
    N3 = N // 4
    pts1 = jnp.transpose(x, (0, 2, 1))[:, :, :3]
    pts1_t = jnp.transpose(pts1, (0, 2, 1))
    x1 = _gscm_first(pts1_t, K, W1)
    fps2 = _fps_pallas(pts1, N2)
    pts2 = jnp.transpose(_gather_op(pts1_t, fps2), (0, 2, 1))
    x1_ds = _gather_op(x1, fps2)
    pts2_t = jnp.transpose(pts2, (0, 2, 1))
    x2 = _gscm(pts2_t, x1_ds, K, W2)
    fps3 = _fps_pallas(pts2, N3)
    pts3 = jnp.transpose(_gather_op(pts2_t, fps3), (0, 2, 1))
    x2_ds = _gather_op(x2, fps3)
    x1_ds = _gather_op(x1_ds, fps3)
    pts3_t = jnp.transpose(pts3, (0, 2, 1))
    x3 = _gscm(pts3_t, x2_ds, K, W3)
    h = jnp.concatenate([x1_ds, x2_ds, x3], axis=1)

    y4, m4, v4 = _conv_stats(h, W4, nb=1024)
    h = _lrelu((y4 - m4[None, :, None]) * jax.lax.rsqrt(v4[None, :, None] + 1e-5))

    dist, idx = _three_nn(pts2, pts3)
    w = 1.0 / (dist + 1e-8)
    w = w / jnp.sum(w, axis=2, keepdims=True)
    h = _three_interpolate(h, idx, w)
    dist, idx = _three_nn(pts1, pts2)
    w = 1.0 / (dist + 1e-8)
    w = w / jnp.sum(w, axis=2, keepdims=True)
    h = _three_interpolate(h, idx, w)

    y5, m5, v5 = _conv_stats(h, W5, nb=1024)
    y6, m6, v6 = _norm_conv_stats(y5, m5, v5, W6, nb=1024)
    out = _norm_conv(y6, m6, v6, W7, nb=1024)
    return out


# speedup vs baseline: 1.5334x; 1.0063x over previous
import functools

import jax
import jax.numpy as jnp
import numpy as np
from jax import lax
from jax.experimental import pallas as pl
from jax.experimental.pallas import tpu as pltpu
from jax.experimental.pallas import tpu_sc as plsc

K = 20


def _sc_gather(table, idx):
    V, D = table.shape
    M = idx.shape[0]
    info = plsc.get_sparse_core_info()
    NC, NS = info.num_cores, info.num_subcores
    NW = NC * NS
    b_per_w = M // NW
    mesh = plsc.VectorSubcoreMesh(core_axis_name="c", subcore_axis_name="s")

    @functools.partial(
        pl.kernel, mesh=mesh,
        out_type=jax.ShapeDtypeStruct((M, D), jnp.float32),
        compiler_params=pltpu.CompilerParams(use_tc_tiling_on_sc=False),
        scratch_types=[
            pltpu.VMEM((b_per_w,), jnp.int32),
            pltpu.VMEM((b_per_w, D), jnp.float32),
            pltpu.SemaphoreType.DMA,
        ],
    )
    def k(table_hbm, idx_hbm, out_hbm, idx_v, rows_v, sem):
        wid = lax.axis_index("s") * NC + lax.axis_index("c")
        base = wid * b_per_w
        pltpu.sync_copy(idx_hbm.at[pl.ds(base, b_per_w)], idx_v)
        pltpu.async_copy(table_hbm.at[idx_v], rows_v, sem).wait()
        pltpu.sync_copy(rows_v, out_hbm.at[pl.ds(base, b_per_w)])

    return k(table, idx)


def _gather_feats_sc(x, idx):
    B, C, N = x.shape
    M = idx.shape[1]
    tab = jnp.transpose(x, (0, 2, 1)).reshape(B * N, C)
    flat = (idx + (jnp.arange(B, dtype=jnp.int32) * N)[:, None]).reshape(B * M)
    rows = _sc_gather(tab, flat)
    return jnp.transpose(rows.reshape(B, M, C), (0, 2, 1))


def _conv_stats_body(h_ref, w_ref, y_ref, s_ref):
    h = h_ref[0]
    w = w_ref[...]
    y = jnp.dot(w, h, preferred_element_type=jnp.float32)
    y_ref[0] = y
    s_ref[0, 0, :] = jnp.sum(y, axis=1)
    s_ref[0, 1, :] = jnp.sum(y * y, axis=1)


def _norm_conv_stats_body(y_ref, m_ref, v_ref, w_ref, z_ref, s_ref):
    y = y_ref[0]
    m = m_ref[...][:, None]
    v = v_ref[...][:, None]
    zn = (y - m) * jax.lax.rsqrt(v + 1e-5)
    zn = jnp.where(zn >= 0, zn, 0.2 * zn)
    z = jnp.dot(w_ref[...], zn, preferred_element_type=jnp.float32)
    z_ref[0] = z
    s_ref[0, 0, :] = jnp.sum(z, axis=1)
    s_ref[0, 1, :] = jnp.sum(z * z, axis=1)


def _norm_conv_body(y_ref, m_ref, v_ref, w_ref, z_ref):
    y = y_ref[0]
    m = m_ref[...][:, None]
    v = v_ref[...][:, None]
    zn = (y - m) * jax.lax.rsqrt(v + 1e-5)
    zn = jnp.where(zn >= 0, zn, 0.2 * zn)
    z_ref[0] = jnp.dot(w_ref[...], zn, preferred_element_type=jnp.float32)


def _conv_stats(h, w, nb=1024):
    B, C, N = h.shape
    O = w.shape[0]
    grid = (B, N // nb)
    y, s = pl.pallas_call(
        _conv_stats_body,
        grid=grid,
        in_specs=[
            pl.BlockSpec((1, C, nb), lambda b, n: (b, 0, n)),
            pl.BlockSpec((O, C), lambda b, n: (0, 0)),
        ],
        out_specs=[
            pl.BlockSpec((1, O, nb), lambda b, n: (b, 0, n)),
            pl.BlockSpec((1, 2, O), lambda b, n: (b * (N // nb) + n, 0, 0)),
        ],
        out_shape=[
            jax.ShapeDtypeStruct((B, O, N), jnp.float32),
            jax.ShapeDtypeStruct((B * (N // nb), 2, O), jnp.float32),
        ],
    )(h, w)
    tot = jnp.sum(s, axis=0)
    cnt = B * N
    mean = tot[0] / cnt
    var = tot[1] / cnt - mean * mean
    return y, mean, var


def _norm_conv_stats(y, mean, var, w, nb=1024):
    B, C, N = y.shape
    O = w.shape[0]
    grid = (B, N // nb)
    z, s = pl.pallas_call(
        _norm_conv_stats_body,
        grid=grid,
        in_specs=[
            pl.BlockSpec((1, C, nb), lambda b, n: (b, 0, n)),
            pl.BlockSpec((C,), lambda b, n: (0,)),
            pl.BlockSpec((C,), lambda b, n: (0,)),
            pl.BlockSpec((O, C), lambda b, n: (0, 0)),
        ],
        out_specs=[
            pl.BlockSpec((1, O, nb), lambda b, n: (b, 0, n)),
            pl.BlockSpec((1, 2, O), lambda b, n: (b * (N // nb) + n, 0, 0)),
        ],
        out_shape=[
            jax.ShapeDtypeStruct((B, O, N), jnp.float32),
            jax.ShapeDtypeStruct((B * (N // nb), 2, O), jnp.float32),
        ],
    )(y, mean, var, w)
    tot = jnp.sum(s, axis=0)
    cnt = B * N
    m = tot[0] / cnt
    v = tot[1] / cnt - m * m
    return z, m, v


def _norm_conv(y, mean, var, w, nb=1024):
    B, C, N = y.shape
    O = w.shape[0]
    grid = (B, N // nb)
    return pl.pallas_call(
        _norm_conv_body,
        grid=grid,
        in_specs=[
            pl.BlockSpec((1, C, nb), lambda b, n: (b, 0, n)),
            pl.BlockSpec((C,), lambda b, n: (0,)),
            pl.BlockSpec((C,), lambda b, n: (0,)),
            pl.BlockSpec((O, C), lambda b, n: (0, 0)),
        ],
        out_specs=pl.BlockSpec((1, O, nb), lambda b, n: (b, 0, n)),
        out_shape=jax.ShapeDtypeStruct((B, O, N), jnp.float32),
    )(y, mean, var, w)


def _eig3_body(cov_ref, eig_ref):
    a00 = cov_ref[0, 0, :]
    a11 = cov_ref[0, 1, :]
    a22 = cov_ref[0, 2, :]
    a01 = cov_ref[0, 3, :]
    a02 = cov_ref[0, 4, :]
    a12 = cov_ref[0, 5, :]
    q = (a00 + a11 + a22) * (1.0 / 3.0)
    p1 = a01 * a01 + a02 * a02 + a12 * a12
    d01 = a00 - a11
    d02 = a00 - a22
    d12 = a11 - a22
    b00 = (d01 + d02) * (1.0 / 3.0)
    b11 = (-d01 + d12) * (1.0 / 3.0)
    b22 = (-d02 - d12) * (1.0 / 3.0)
    p2 = b00 * b00 + b11 * b11 + b22 * b22 + 2.0 * p1
    p = jnp.sqrt(p2 * (1.0 / 6.0) + 1e-30)
    detb = (b00 * (b11 * b22 - a12 * a12)
            - a01 * (a01 * b22 - a12 * a02)
            + a02 * (a01 * a12 - b11 * a02))
    r = detb / (2.0 * p * p * p)
    r = jnp.clip(r, -1.0, 1.0)

    def newton_root(rr):
        xx = jnp.ones_like(rr)
        for _ in range(25):
            g = (4.0 * xx * xx - 3.0) * xx - rr
            dg = jnp.maximum(12.0 * xx * xx - 3.0, 1e-8)
            xx = xx - g / dg
        return xx

    lmax = q + 2.0 * p * newton_root(r)
    lmin = q - 2.0 * p * newton_root(-r)
    lmid = 3.0 * q - lmax - lmin
    eig_ref[0, 0, :] = lmin
    eig_ref[0, 1, :] = lmid
    eig_ref[0, 2, :] = lmax


def _eig3(cov6):
    B, _, N = cov6.shape
    nb = min(N, 2048)
    return pl.pallas_call(
        _eig3_body,
        grid=(B, N // nb),
        in_specs=[pl.BlockSpec((1, 6, nb), lambda b, n: (b, 0, n))],
        out_specs=pl.BlockSpec((1, 3, nb), lambda b, n: (b, 0, n)),
        out_shape=jax.ShapeDtypeStruct((B, 3, N), jnp.float32),
    )(cov6)


def _fps_body(x_ref, y_ref, z_ref, idx_ref, npoint, rows, cols):
    X = x_ref[0]
    Y = y_ref[0]
    Z = z_ref[0]
    flat_i = (jax.lax.broadcasted_iota(jnp.int32, (rows, cols), 0) * cols
              + jax.lax.broadcasted_iota(jnp.int32, (rows, cols), 1))
    ocols = npoint // 8
    flat_o = (jax.lax.broadcasted_iota(jnp.int32, (8, ocols), 0) * ocols
              + jax.lax.broadcasted_iota(jnp.int32, (8, ocols), 1))

    def body(i, state):
        dists, far, acc = state
        acc = jnp.where(flat_o == i, far, acc)
        sel = flat_i == far
        cx = jnp.sum(jnp.where(sel, X, 0.0))
        cy = jnp.sum(jnp.where(sel, Y, 0.0))
        cz = jnp.sum(jnp.where(sel, Z, 0.0))
        dx = X - cx
        dy = Y - cy
        dz = Z - cz
        d = dx * dx + dy * dy + dz * dz
        dists = jnp.minimum(dists, d)
        m = jnp.max(dists)
        far2 = jnp.min(jnp.where(dists == m, flat_i, jnp.int32(2 ** 30)))
        return dists, far2, acc

    d0 = jnp.full((rows, cols), 1e10, dtype=jnp.float32)
    acc0 = jnp.zeros((8, ocols), dtype=jnp.int32)
    _, _, acc = jax.lax.fori_loop(0, npoint, body, (d0, jnp.int32(0), acc0))
    idx_ref[0] = acc


def _fps_pallas(pts, npoint):
    B, N, _ = pts.shape
    rows, cols = 8, N // 8
    xyz = jnp.transpose(pts, (0, 2, 1)).reshape(B, 3, rows, cols)
    body = functools.partial(_fps_body, npoint=npoint, rows=rows, cols=cols)
    return pl.pallas_call(
        body,
        grid=(B,),
        in_specs=[
            pl.BlockSpec((1, rows, cols), lambda b: (b, 0, 0))
            for _ in range(3)
        ],
        out_specs=pl.BlockSpec((1, 8, npoint // 8), lambda b: (b, 0, 0)),
        out_shape=jax.ShapeDtypeStruct((B, 8, npoint // 8), jnp.int32),
    )(xyz[:, 0], xyz[:, 1], xyz[:, 2]).reshape(B, npoint)


def _topk_body(rows_ref, cols_ref, idx_ref, val_ref, k, clamp):
    pb = rows_ref[0]
    pt = cols_ref[0]
    R = pb.shape[0]
    N = pt.shape[1]
    xxr = pb[:, 0] * pb[:, 0] + pb[:, 1] * pb[:, 1] + pb[:, 2] * pb[:, 2]
    xxm = pt[0, :] * pt[0, :] + pt[1, :] * pt[1, :] + pt[2, :] * pt[2, :]
    inner = jnp.dot(pb, pt, preferred_element_type=jnp.float32)
    if clamp:
        d2 = jnp.maximum((xxr[:, None] + xxm[None, :]) - 2.0 * inner, 0.0)
    else:
        d2 = (xxr[:, None] - 2.0 * inner) + xxm[None, :]
    ci = jax.lax.broadcasted_iota(jnp.int32, (R, N), 1)
    cols = []
    vals = []
    for _ in range(k):
        m = jnp.min(d2, axis=1)
        eq = d2 == m[:, None]
        col = jnp.min(jnp.where(eq, ci, jnp.int32(N)), axis=1)
        cols.append(col)
        vals.append(m)
        d2 = jnp.where(ci == col[:, None], jnp.float32(jnp.inf), d2)
    idx_ref[0] = jnp.stack(cols, axis=1)
    val_ref[0] = jnp.stack(vals, axis=1)


def _topk_pallas(rows, cols, k, clamp, rblk=512):
    B, Nr, _ = rows.shape
    Nc = cols.shape[2]
    rb = min(rblk, Nr)
    body = functools.partial(_topk_body, k=k, clamp=clamp)
    return pl.pallas_call(
        body,
        grid=(B, Nr // rb),
        in_specs=[
            pl.BlockSpec((1, rb, 3), lambda b, n: (b, n, 0)),
            pl.BlockSpec((1, 3, Nc), lambda b, n: (b, 0, 0)),
        ],
        out_specs=[
            pl.BlockSpec((1, rb, k), lambda b, n: (b, n, 0)),
            pl.BlockSpec((1, rb, k), lambda b, n: (b, n, 0)),
        ],
        out_shape=[
            jax.ShapeDtypeStruct((B, Nr, k), jnp.int32),
            jax.ShapeDtypeStruct((B, Nr, k), jnp.float32),
        ],
    )(rows, cols)


def _sel_body(s_ref, idx_ref, k):
    s = s_ref[0]
    R, N = s.shape
    ci = jax.lax.broadcasted_iota(jnp.int32, (R, N), 1)
    cols = []
    for _ in range(k):
        m = jnp.max(s, axis=1)
        col = jnp.min(jnp.where(s == m[:, None], ci, jnp.int32(N)), axis=1)
        cols.append(col)
        s = jnp.where(ci == col[:, None], -jnp.float32(jnp.inf), s)
    idx_ref[0] = jnp.stack(cols, axis=1)


def _topk_select(scores, k, rblk=512):
    B, Nr, Nc = scores.shape
    rb = min(rblk, Nr)
    return pl.pallas_call(
        functools.partial(_sel_body, k=k),
        grid=(B, Nr // rb),
        in_specs=[pl.BlockSpec((1, rb, Nc), lambda b, n: (b, n, 0))],
        out_specs=pl.BlockSpec((1, rb, k), lambda b, n: (b, n, 0)),
        out_shape=jax.ShapeDtypeStruct((B, Nr, k), jnp.int32),
    )(scores)


def _knn_exact(x, k):
    xx = jnp.sum(x * x, axis=1)
    inner = jnp.einsum('bcn,bcm->bnm', x, x)
    neg_d2 = -(xx[:, :, None] - 2.0 * inner + xx[:, None, :])
    return _topk_select(neg_d2, k)


def _knn(x, k):
    rows = jnp.transpose(x, (0, 2, 1))
    return _topk_pallas(rows, x, k, clamp=False)[0]


def _knn_with_d2(x, k):
    rows = jnp.transpose(x, (0, 2, 1))
    return _topk_pallas(rows, x, k, clamp=False)


def _gather_nb(x, idx):
    return jax.vmap(lambda xb, ib: xb[:, ib])(x, idx)


def _gather_op(x, idx):
    return jax.vmap(lambda xb, ib: xb[:, ib])(x, idx)


def _group_layer(x, idx):
    nb = _gather_nb(x, idx)
    xc = jnp.broadcast_to(x[:, :, :, None], nb.shape)
    return jnp.concatenate([nb - xc, xc], axis=1)


def _eigen_graph(x, k):
    idx_EU = _knn_exact(x, k)
    nb = _gather_nb(x, idx_EU)
    nb = jnp.transpose(nb, (0, 2, 3, 1))
    nb = nb - jnp.mean(nb, axis=2, keepdims=True)
    cov = jnp.einsum('bnki,bnkj->bnij', nb, nb) / k
    eig = jnp.linalg.eigvalsh(jax.lax.stop_gradient(cov))
    idx_EI = _knn_exact(jnp.transpose(eig, (0, 2, 1)), k)
    return x, idx_EU, idx_EI


def _graph_distance(x, idx):
    nb = _gather_nb(x, idx)
    return jnp.sqrt(jnp.sum((nb - x[:, :, :, None]) ** 2, axis=1, keepdims=True) + 1e-12)


def _bn(x, axes):
    m = jnp.mean(x, axis=axes, keepdims=True)
    v = jnp.var(x, axis=axes, keepdims=True)
    return (x - m) / jnp.sqrt(v + 1e-5)


def _lrelu(x):
    return jax.nn.leaky_relu(x, 0.2)


def _conv2d(W, x):
    return jnp.einsum('oc,bcnk->bonk', W, x)


def _fps(xyz, npoint):
    xyz = jax.lax.stop_gradient(xyz)
    B, N, _ = xyz.shape
    def body(i, state):
        idxs, dists, far = state
        idxs = idxs.at[:, i].set(far)
        centroid = jnp.take_along_axis(xyz, far[:, None, None], axis=1)
        d = jnp.sum((xyz - centroid) ** 2, axis=-1)
        dists = jnp.minimum(dists, d)
        far = jnp.argmax(dists, axis=-1).astype(jnp.int32)
        return idxs, dists, far
    idxs0 = jnp.zeros((B, npoint), dtype=jnp.int32)
    d0 = jnp.full((B, N), 1e10, dtype=xyz.dtype)
    f0 = jnp.zeros((B,), dtype=jnp.int32)
    idxs, _, _ = jax.lax.fori_loop(0, npoint, body, (idxs0, d0, f0))
    return idxs


def _three_nn(unknown, known):
    idx, d2 = _topk_pallas(unknown, jnp.transpose(known, (0, 2, 1)), 3,
                           clamp=True)
    dist = jnp.sqrt(jnp.maximum(d2, 0.0) + 1e-12)
    return dist, idx


def _three_interpolate(feats, idx, weight):
    def one(fb, ib, wb):
        nb = fb[:, ib]
        return jnp.sum(nb * wb[None], axis=-1)
    return jax.vmap(one)(feats, idx, weight)


def _gscm_first(points_t, k, W):
    xg, idx_EU, idx_EI = _eigen_graph(points_t, k)
    feat = jnp.concatenate([_group_layer(xg, idx_EU), _group_layer(xg, idx_EI)], axis=1)
    dist = _graph_distance(points_t, idx_EU)
    feat = jnp.concatenate([feat, dist], axis=1)
    y = _lrelu(_bn(_conv2d(W, feat), (0, 2, 3)))
    return jnp.max(y, axis=-1)


def _gscm(points_t, feats, k, W):
    _, idx_EU, idx_EI = _eigen_graph(points_t, k)
    feat = jnp.concatenate([_group_layer(feats, idx_EU), _group_layer(feats, idx_EI)], axis=1)
    y = _lrelu(_bn(_conv2d(W, feat), (0, 2, 3)))
    return jnp.max(y, axis=-1)


def kernel(x, W1, W2, W3, W4, W5, W6, W7):
    B, C, N = x.shape
    N2 = N // 2
    N3 = N // 4
    pts1 = jnp.transpose(x, (0, 2, 1))[:, :, :3]
    pts1_t = jnp.transpose(pts1, (0, 2, 1))
    x1 = _gscm_first(pts1_t, K, W1)
    fps2 = _fps_pallas(pts1, N2)
    pts2 = jnp.transpose(_gather_op(pts1_t, fps2), (0, 2, 1))
    x1_ds = _gather_feats_sc(x1, fps2)
    pts2_t = jnp.transpose(pts2, (0, 2, 1))
    x2 = _gscm(pts2_t, x1_ds, K, W2)
    fps3 = _fps_pallas(pts2, N3)
    pts3 = jnp.transpose(_gather_op(pts2_t, fps3), (0, 2, 1))
    x2_ds = _gather_feats_sc(x2, fps3)
    x1_ds = _gather_feats_sc(x1_ds, fps3)
    pts3_t = jnp.transpose(pts3, (0, 2, 1))
    x3 = _gscm(pts3_t, x2_ds, K, W3)
    h = jnp.concatenate([x1_ds, x2_ds, x3], axis=1)

    y4, m4, v4 = _conv_stats(h, W4, nb=1024)
    h = _lrelu((y4 - m4[None, :, None]) * jax.lax.rsqrt(v4[None, :, None] + 1e-5))

    dist, idx = _three_nn(pts2, pts3)
    w = 1.0 / (dist + 1e-8)
    w = w / jnp.sum(w, axis=2, keepdims=True)
    h = _three_interpolate(h, idx, w)
    dist, idx = _three_nn(pts1, pts2)
    w = 1.0 / (dist + 1e-8)
    w = w / jnp.sum(w, axis=2, keepdims=True)
    h = _three_interpolate(h, idx, w)

    y5, m5, v5 = _conv_stats(h, W5, nb=1024)
    y6, m6, v6 = _norm_conv_stats(y5, m5, v5, W6, nb=1024)
    out = _norm_conv(y6, m6, v6, W7, nb=1024)
    return out


# speedup vs baseline: 1.5335x; 1.0000x over previous
import functools

import jax
import jax.numpy as jnp
from jax import lax
from jax.experimental import pallas as pl
from jax.experimental.pallas import tpu as pltpu
from jax.experimental.pallas import tpu_sc as plsc

K = 20


def _sc_gather(table, idx):
    V, D = table.shape
    M = idx.shape[0]
    info = plsc.get_sparse_core_info()
    NC, NS = info.num_cores, info.num_subcores
    NW = NC * NS
    b_per_w = M // NW
    mesh = plsc.VectorSubcoreMesh(core_axis_name="c", subcore_axis_name="s")

    @functools.partial(
        pl.kernel, mesh=mesh,
        out_type=jax.ShapeDtypeStruct((M, D), jnp.float32),
        compiler_params=pltpu.CompilerParams(use_tc_tiling_on_sc=False),
        scratch_types=[
            pltpu.VMEM((b_per_w,), jnp.int32),
            pltpu.VMEM((b_per_w, D), jnp.float32),
            pltpu.SemaphoreType.DMA,
        ],
    )
    def k(table_hbm, idx_hbm, out_hbm, idx_v, rows_v, sem):
        wid = lax.axis_index("s") * NC + lax.axis_index("c")
        base = wid * b_per_w
        pltpu.sync_copy(idx_hbm.at[pl.ds(base, b_per_w)], idx_v)
        pltpu.async_copy(table_hbm.at[idx_v], rows_v, sem).wait()
        pltpu.sync_copy(rows_v, out_hbm.at[pl.ds(base, b_per_w)])

    return k(table, idx)


def _gather_feats_sc(x, idx):
    B, C, N = x.shape
    M = idx.shape[1]
    tab = jnp.transpose(x, (0, 2, 1)).reshape(B * N, C)
    flat = (idx + (jnp.arange(B, dtype=jnp.int32) * N)[:, None]).reshape(B * M)
    rows = _sc_gather(tab, flat)
    return jnp.transpose(rows.reshape(B, M, C), (0, 2, 1))


def _conv_stats_body(h_ref, w_ref, y_ref, s_ref):
    h = h_ref[0]
    w = w_ref[...]
    y = jnp.dot(w, h, preferred_element_type=jnp.float32)
    y_ref[0] = y
    s_ref[0, 0, :] = jnp.sum(y, axis=1)
    s_ref[0, 1, :] = jnp.sum(y * y, axis=1)


def _norm_conv_stats_body(y_ref, m_ref, v_ref, w_ref, z_ref, s_ref):
    y = y_ref[0]
    m = m_ref[...][:, None]
    v = v_ref[...][:, None]
    zn = (y - m) * jax.lax.rsqrt(v + 1e-5)
    zn = jnp.where(zn >= 0, zn, 0.2 * zn)
    z = jnp.dot(w_ref[...], zn, preferred_element_type=jnp.float32)
    z_ref[0] = z
    s_ref[0, 0, :] = jnp.sum(z, axis=1)
    s_ref[0, 1, :] = jnp.sum(z * z, axis=1)


def _norm_conv_body(y_ref, m_ref, v_ref, w_ref, z_ref):
    y = y_ref[0]
    m = m_ref[...][:, None]
    v = v_ref[...][:, None]
    zn = (y - m) * jax.lax.rsqrt(v + 1e-5)
    zn = jnp.where(zn >= 0, zn, 0.2 * zn)
    z_ref[0] = jnp.dot(w_ref[...], zn, preferred_element_type=jnp.float32)


def _conv_stats(h, w, nb=1024):
    B, C, N = h.shape
    O = w.shape[0]
    grid = (B, N // nb)
    y, s = pl.pallas_call(
        _conv_stats_body,
        grid=grid,
        in_specs=[
            pl.BlockSpec((1, C, nb), lambda b, n: (b, 0, n)),
            pl.BlockSpec((O, C), lambda b, n: (0, 0)),
        ],
        out_specs=[
            pl.BlockSpec((1, O, nb), lambda b, n: (b, 0, n)),
            pl.BlockSpec((1, 2, O), lambda b, n: (b * (N // nb) + n, 0, 0)),
        ],
        out_shape=[
            jax.ShapeDtypeStruct((B, O, N), jnp.float32),
            jax.ShapeDtypeStruct((B * (N // nb), 2, O), jnp.float32),
        ],
    )(h, w)
    tot = jnp.sum(s, axis=0)
    cnt = B * N
    mean = tot[0] / cnt
    var = tot[1] / cnt - mean * mean
    return y, mean, var


def _norm_conv_stats(y, mean, var, w, nb=1024):
    B, C, N = y.shape
    O = w.shape[0]
    grid = (B, N // nb)
    z, s = pl.pallas_call(
        _norm_conv_stats_body,
        grid=grid,
        in_specs=[
            pl.BlockSpec((1, C, nb), lambda b, n: (b, 0, n)),
            pl.BlockSpec((C,), lambda b, n: (0,)),
            pl.BlockSpec((C,), lambda b, n: (0,)),
            pl.BlockSpec((O, C), lambda b, n: (0, 0)),
        ],
        out_specs=[
            pl.BlockSpec((1, O, nb), lambda b, n: (b, 0, n)),
            pl.BlockSpec((1, 2, O), lambda b, n: (b * (N // nb) + n, 0, 0)),
        ],
        out_shape=[
            jax.ShapeDtypeStruct((B, O, N), jnp.float32),
            jax.ShapeDtypeStruct((B * (N // nb), 2, O), jnp.float32),
        ],
    )(y, mean, var, w)
    tot = jnp.sum(s, axis=0)
    cnt = B * N
    m = tot[0] / cnt
    v = tot[1] / cnt - m * m
    return z, m, v


def _norm_conv(y, mean, var, w, nb=1024):
    B, C, N = y.shape
    O = w.shape[0]
    grid = (B, N // nb)
    return pl.pallas_call(
        _norm_conv_body,
        grid=grid,
        in_specs=[
            pl.BlockSpec((1, C, nb), lambda b, n: (b, 0, n)),
            pl.BlockSpec((C,), lambda b, n: (0,)),
            pl.BlockSpec((C,), lambda b, n: (0,)),
            pl.BlockSpec((O, C), lambda b, n: (0, 0)),
        ],
        out_specs=pl.BlockSpec((1, O, nb), lambda b, n: (b, 0, n)),
        out_shape=jax.ShapeDtypeStruct((B, O, N), jnp.float32),
    )(y, mean, var, w)


def _fps_body(x_ref, y_ref, z_ref, idx_ref, npoint, rows, cols):
    X = x_ref[0]
    Y = y_ref[0]
    Z = z_ref[0]
    flat_i = (jax.lax.broadcasted_iota(jnp.int32, (rows, cols), 0) * cols
              + jax.lax.broadcasted_iota(jnp.int32, (rows, cols), 1))
    ocols = npoint // 8
    flat_o = (jax.lax.broadcasted_iota(jnp.int32, (8, ocols), 0) * ocols
              + jax.lax.broadcasted_iota(jnp.int32, (8, ocols), 1))

    def body(i, state):
        dists, far, acc = state
        acc = jnp.where(flat_o == i, far, acc)
        sel = flat_i == far
        cx = jnp.sum(jnp.where(sel, X, 0.0))
        cy = jnp.sum(jnp.where(sel, Y, 0.0))
        cz = jnp.sum(jnp.where(sel, Z, 0.0))
        dx = X - cx
        dy = Y - cy
        dz = Z - cz
        d = dx * dx + dy * dy + dz * dz
        dists = jnp.minimum(dists, d)
        m = jnp.max(dists)
        far2 = jnp.min(jnp.where(dists == m, flat_i, jnp.int32(2 ** 30)))
        return dists, far2, acc

    d0 = jnp.full((rows, cols), 1e10, dtype=jnp.float32)
    acc0 = jnp.zeros((8, ocols), dtype=jnp.int32)
    _, _, acc = jax.lax.fori_loop(0, npoint, body, (d0, jnp.int32(0), acc0))
    idx_ref[0] = acc


def _fps_pallas(pts, npoint):
    B, N, _ = pts.shape
    rows, cols = 8, N // 8
    xyz = jnp.transpose(pts, (0, 2, 1)).reshape(B, 3, rows, cols)
    body = functools.partial(_fps_body, npoint=npoint, rows=rows, cols=cols)
    return pl.pallas_call(
        body,
        grid=(B,),
        in_specs=[
            pl.BlockSpec((1, rows, cols), lambda b: (b, 0, 0))
            for _ in range(3)
        ],
        out_specs=pl.BlockSpec((1, 8, npoint // 8), lambda b: (b, 0, 0)),
        out_shape=jax.ShapeDtypeStruct((B, 8, npoint // 8), jnp.int32),
    )(xyz[:, 0], xyz[:, 1], xyz[:, 2]).reshape(B, npoint)


def _topk_body(rows_ref, cols_ref, idx_ref, val_ref, k, clamp):
    pb = rows_ref[0]
    pt = cols_ref[0]
    R = pb.shape[0]
    N = pt.shape[1]
    xxr = pb[:, 0] * pb[:, 0] + pb[:, 1] * pb[:, 1] + pb[:, 2] * pb[:, 2]
    xxm = pt[0, :] * pt[0, :] + pt[1, :] * pt[1, :] + pt[2, :] * pt[2, :]
    inner = jnp.dot(pb, pt, preferred_element_type=jnp.float32)
    if clamp:
        d2 = jnp.maximum((xxr[:, None] + xxm[None, :]) - 2.0 * inner, 0.0)
    else:
        d2 = (xxr[:, None] - 2.0 * inner) + xxm[None, :]
    ci = jax.lax.broadcasted_iota(jnp.int32, (R, N), 1)
    cols = []
    vals = []
    for _ in range(k):
        m = jnp.min(d2, axis=1)
        eq = d2 == m[:, None]
        col = jnp.min(jnp.where(eq, ci, jnp.int32(N)), axis=1)
        cols.append(col)
        vals.append(m)
        d2 = jnp.where(ci == col[:, None], jnp.float32(jnp.inf), d2)
    idx_ref[0] = jnp.stack(cols, axis=1)
    val_ref[0] = jnp.stack(vals, axis=1)


def _topk_pallas(rows, cols, k, clamp, rblk=512):
    B, Nr, _ = rows.shape
    Nc = cols.shape[2]
    rb = min(rblk, Nr)
    body = functools.partial(_topk_body, k=k, clamp=clamp)
    return pl.pallas_call(
        body,
        grid=(B, Nr // rb),
        in_specs=[
            pl.BlockSpec((1, rb, 3), lambda b, n: (b, n, 0)),
            pl.BlockSpec((1, 3, Nc), lambda b, n: (b, 0, 0)),
        ],
        out_specs=[
            pl.BlockSpec((1, rb, k), lambda b, n: (b, n, 0)),
            pl.BlockSpec((1, rb, k), lambda b, n: (b, n, 0)),
        ],
        out_shape=[
            jax.ShapeDtypeStruct((B, Nr, k), jnp.int32),
            jax.ShapeDtypeStruct((B, Nr, k), jnp.float32),
        ],
    )(rows, cols)


def _sel_body(s_ref, idx_ref, k):
    s = s_ref[0]
    R, N = s.shape
    ci = jax.lax.broadcasted_iota(jnp.int32, (R, N), 1)
    cols = []
    for _ in range(k):
        m = jnp.max(s, axis=1)
        col = jnp.min(jnp.where(s == m[:, None], ci, jnp.int32(N)), axis=1)
        cols.append(col)
        s = jnp.where(ci == col[:, None], -jnp.float32(jnp.inf), s)
    idx_ref[0] = jnp.stack(cols, axis=1)


def _topk_select(scores, k, rblk=512):
    B, Nr, Nc = scores.shape
    rb = min(rblk, Nr)
    return pl.pallas_call(
        functools.partial(_sel_body, k=k),
        grid=(B, Nr // rb),
        in_specs=[pl.BlockSpec((1, rb, Nc), lambda b, n: (b, n, 0))],
        out_specs=pl.BlockSpec((1, rb, k), lambda b, n: (b, n, 0)),
        out_shape=jax.ShapeDtypeStruct((B, Nr, k), jnp.int32),
    )(scores)


def _knn_exact(x, k):
    xx = jnp.sum(x * x, axis=1)
    inner = jnp.einsum('bcn,bcm->bnm', x, x)
    neg_d2 = -(xx[:, :, None] - 2.0 * inner + xx[:, None, :])
    return _topk_select(neg_d2, k)


def _gather_nb(x, idx):
    return jax.vmap(lambda xb, ib: xb[:, ib])(x, idx)


def _gather_op(x, idx):
    return jax.vmap(lambda xb, ib: xb[:, ib])(x, idx)


def _group_layer(x, idx):
    nb = _gather_nb(x, idx)
    xc = jnp.broadcast_to(x[:, :, :, None], nb.shape)
    return jnp.concatenate([nb - xc, xc], axis=1)


def _eigen_graph(x, k):
    idx_EU = _knn_exact(x, k)
    nb = _gather_nb(x, idx_EU)
    nb = jnp.transpose(nb, (0, 2, 3, 1))
    nb = nb - jnp.mean(nb, axis=2, keepdims=True)
    cov = jnp.einsum('bnki,bnkj->bnij', nb, nb) / k
    eig = jnp.linalg.eigvalsh(jax.lax.stop_gradient(cov))
    idx_EI = _knn_exact(jnp.transpose(eig, (0, 2, 1)), k)
    return x, idx_EU, idx_EI


def _graph_distance(x, idx):
    nb = _gather_nb(x, idx)
    return jnp.sqrt(jnp.sum((nb - x[:, :, :, None]) ** 2, axis=1, keepdims=True) + 1e-12)


def _bn(x, axes):
    m = jnp.mean(x, axis=axes, keepdims=True)
    v = jnp.var(x, axis=axes, keepdims=True)
    return (x - m) / jnp.sqrt(v + 1e-5)


def _lrelu(x):
    return jax.nn.leaky_relu(x, 0.2)


def _conv2d(W, x):
    return jnp.einsum('oc,bcnk->bonk', W, x)


def _three_nn(unknown, known):
    idx, d2 = _topk_pallas(unknown, jnp.transpose(known, (0, 2, 1)), 3,
                           clamp=True)
    dist = jnp.sqrt(jnp.maximum(d2, 0.0) + 1e-12)
    return dist, idx


def _three_interpolate(feats, idx, weight):
    def one(fb, ib, wb):
        nb = fb[:, ib]
        return jnp.sum(nb * wb[None], axis=-1)
    return jax.vmap(one)(feats, idx, weight)


def _gscm_first(points_t, k, W):
    xg, idx_EU, idx_EI = _eigen_graph(points_t, k)
    feat = jnp.concatenate([_group_layer(xg, idx_EU), _group_layer(xg, idx_EI)], axis=1)
    dist = _graph_distance(points_t, idx_EU)
    feat = jnp.concatenate([feat, dist], axis=1)
    y = _lrelu(_bn(_conv2d(W, feat), (0, 2, 3)))
    return jnp.max(y, axis=-1)


def _gscm(points_t, feats, k, W):
    _, idx_EU, idx_EI = _eigen_graph(points_t, k)
    feat = jnp.concatenate([_group_layer(feats, idx_EU), _group_layer(feats, idx_EI)], axis=1)
    y = _lrelu(_bn(_conv2d(W, feat), (0, 2, 3)))
    return jnp.max(y, axis=-1)


def kernel(x, W1, W2, W3, W4, W5, W6, W7):
    B, C, N = x.shape
    N2 = N // 2
    N3 = N // 4
    pts1 = jnp.transpose(x, (0, 2, 1))[:, :, :3]
    pts1_t = jnp.transpose(pts1, (0, 2, 1))
    x1 = _gscm_first(pts1_t, K, W1)
    fps2 = _fps_pallas(pts1, N2)
    pts2 = jnp.transpose(_gather_op(pts1_t, fps2), (0, 2, 1))
    x1_ds = _gather_feats_sc(x1, fps2)
    pts2_t = jnp.transpose(pts2, (0, 2, 1))
    x2 = _gscm(pts2_t, x1_ds, K, W2)
    fps3 = _fps_pallas(pts2, N3)
    pts3 = jnp.transpose(_gather_op(pts2_t, fps3), (0, 2, 1))
    x2_ds = _gather_feats_sc(x2, fps3)
    x1_ds = _gather_feats_sc(x1_ds, fps3)
    pts3_t = jnp.transpose(pts3, (0, 2, 1))
    x3 = _gscm(pts3_t, x2_ds, K, W3)
    h = jnp.concatenate([x1_ds, x2_ds, x3], axis=1)

    y4, m4, v4 = _conv_stats(h, W4, nb=1024)
    h = _lrelu((y4 - m4[None, :, None]) * jax.lax.rsqrt(v4[None, :, None] + 1e-5))

    dist, idx = _three_nn(pts2, pts3)
    w = 1.0 / (dist + 1e-8)
    w = w / jnp.sum(w, axis=2, keepdims=True)
    h = _three_interpolate(h, idx, w)
    dist, idx = _three_nn(pts1, pts2)
    w = 1.0 / (dist + 1e-8)
    w = w / jnp.sum(w, axis=2, keepdims=True)
    h = _three_interpolate(h, idx, w)

    y5, m5, v5 = _conv_stats(h, W5, nb=1024)
    y6, m6, v6 = _norm_conv_stats(y5, m5, v5, W6, nb=1024)
    out = _norm_conv(y6, m6, v6, W7, nb=1024)
    return out
